# baseline (speedup 1.0000x reference)
import jax
import jax.numpy as jnp
from jax.experimental import pallas as pl
from jax.experimental.pallas import tpu as pltpu

_NBIN = 25
_DELTA = 2.0 / (_NBIN - 1)
_EPS = 1e-8
_B = 16
_D = 100000
_M = 128
_DBLK = 10000
_ND = _D // _DBLK


def _hist_kernel(q_ref, dx_ref, lab_ref, hlab_ref, hall_ref, lsum_ref):
    j = pl.program_id(1)
    dx = dx_ref[0]
    q = q_ref[0]
    lab = lab_ref[0]

    dots = jax.lax.dot_general(
        dx, q, (((1,), (1,)), ((), ())),
        preferred_element_type=jnp.float32)
    sq = jnp.sum(dx * dx, axis=1, keepdims=True)
    qsq = jnp.sum(q * q, axis=1, keepdims=True)
    denom = jnp.maximum(jnp.sqrt(qsq * sq), _EPS)
    sim = dots / denom

    centers = 1.0 - jax.lax.broadcasted_iota(
        jnp.int32, (1, _NBIN), 1).astype(jnp.float32) * _DELTA
    w = jnp.maximum(1.0 - jnp.abs(sim - centers) * (1.0 / _DELTA), 0.0)
    part_all = jnp.sum(w, axis=0, keepdims=True)
    part_lab = jnp.sum(w * lab, axis=0, keepdims=True)
    part_l = jnp.sum(lab, axis=0, keepdims=True)

    @pl.when(j == 0)
    def _():
        hlab_ref[0] = part_lab
        hall_ref[0] = part_all
        lsum_ref[0] = part_l

    @pl.when(j > 0)
    def _():
        hlab_ref[0] += part_lab
        hall_ref[0] += part_all
        lsum_ref[0] += part_l


def _loss_kernel(hlab_ref, hall_ref, lsum_ref, out_ref):
    hlab = hlab_ref[:, 0, :]
    hall = hall_ref[:, 0, :]
    lsum = lsum_ref[:, 0, :]
    r = jax.lax.broadcasted_iota(jnp.int32, (_NBIN, _NBIN), 0)
    c = jax.lax.broadcasted_iota(jnp.int32, (_NBIN, _NBIN), 1)
    upper = jnp.where(r <= c, 1.0, 0.0)
    cum_lab = jnp.dot(hlab, upper, preferred_element_type=jnp.float32)
    cum_all = jnp.dot(hall, upper, preferred_element_type=jnp.float32) + 1e-16
    precision = cum_lab / cum_all
    recall = hlab / lsum
    pr = precision * recall
    tot = jnp.sum(jnp.sum(pr, axis=0, keepdims=True), axis=1, keepdims=True)
    out_ref[...] = tot * (1.0 / (_B * _NBIN))


@jax.jit
def kernel(qX, dXs, labels):
    labf = labels.astype(jnp.float32).reshape(_B, _D, 1)
    qr = qX.reshape(_B, 1, _M)
    hlab, hall, lsum = pl.pallas_call(
        _hist_kernel,
        grid=(_B, _ND),
        in_specs=[
            pl.BlockSpec((1, 1, _M), lambda b, j: (b, 0, 0)),
            pl.BlockSpec((1, _DBLK, _M), lambda b, j: (b, j, 0)),
            pl.BlockSpec((1, _DBLK, 1), lambda b, j: (b, j, 0)),
        ],
        out_specs=[
            pl.BlockSpec((1, 1, _NBIN), lambda b, j: (b, 0, 0)),
            pl.BlockSpec((1, 1, _NBIN), lambda b, j: (b, 0, 0)),
            pl.BlockSpec((1, 1, 1), lambda b, j: (b, 0, 0)),
        ],
        out_shape=[
            jax.ShapeDtypeStruct((_B, 1, _NBIN), jnp.float32),
            jax.ShapeDtypeStruct((_B, 1, _NBIN), jnp.float32),
            jax.ShapeDtypeStruct((_B, 1, 1), jnp.float32),
        ],
        compiler_params=pltpu.CompilerParams(
            dimension_semantics=("parallel", "arbitrary"),
        ),
    )(qr, dXs, labf)
    out = pl.pallas_call(
        _loss_kernel,
        out_shape=jax.ShapeDtypeStruct((1, 1), jnp.float32),
    )(hlab, hall, lsum)
    return out[0, 0]
